# baseline (speedup 1.0000x reference)
import jax
import jax.numpy as jnp
from jax.experimental import pallas as pl

NC = 1000
SB = 1


CB = 200


def _onehot_block(x_ref, o_ref):
    j = pl.program_id(1)
    idx = x_ref[0]
    iota = jax.lax.broadcasted_iota(
        jnp.int32, (idx.shape[0], CB, idx.shape[1]), 1) + j * CB
    o_ref[...] = (iota == idx[:, None, :]).astype(jnp.int32)


def kernel(x):
    B, S = x.shape
    x3 = x.T.reshape(S // SB, SB, B)
    out_t = pl.pallas_call(
        _onehot_block,
        grid=(S // SB, NC // CB),
        in_specs=[pl.BlockSpec((1, SB, B), lambda i, j: (i, 0, 0))],
        out_specs=pl.BlockSpec((SB, CB, B), lambda i, j: (i, j, 0)),
        out_shape=jax.ShapeDtypeStruct((S, NC, B), jnp.int32),
    )(x3)
    return out_t.transpose(2, 0, 1)


# speedup vs baseline: 2.1246x; 2.1246x over previous
import jax
import jax.numpy as jnp
from jax.experimental import pallas as pl

NC = 1000
SB = 1


def _onehot_block(x_ref, o_ref):
    idx = x_ref[0]
    iota = jax.lax.broadcasted_iota(
        jnp.int32, (idx.shape[0], NC, idx.shape[1]), 1)
    o_ref[...] = (iota == idx[:, None, :]).astype(jnp.int32)


def kernel(x):
    B, S = x.shape
    x3 = x.T.reshape(S // SB, SB, B)
    out_t = pl.pallas_call(
        _onehot_block,
        grid=(S // SB,),
        in_specs=[pl.BlockSpec((1, SB, B), lambda i: (i, 0, 0))],
        out_specs=pl.BlockSpec((SB, NC, B), lambda i: (i, 0, 0)),
        out_shape=jax.ShapeDtypeStruct((S, NC, B), jnp.int32),
    )(x3)
    return out_t.transpose(2, 0, 1)
